# Initial kernel scaffold; baseline (speedup 1.0000x reference)
import jax
import jax.numpy as jnp
from jax.experimental import pallas as pl


def kernel(embedded, symbol, pe):
    raise NotImplementedError("write your pallas kernel here")


# speedup vs baseline: 1.6200x; 1.6200x over previous
import jax
import jax.numpy as jnp
from jax.experimental import pallas as pl


def _body(sym_ref, emb_ref, pe_ref, out_ref):
    mask = (sym_ref[0] != 0).astype(jnp.float32)
    out_ref[0] = emb_ref[0] + pe_ref[...] * mask


def kernel(embedded, symbol, pe):
    B, S, D = embedded.shape
    BS = 512
    n_s = S // BS
    sym3 = symbol.astype(jnp.int32).reshape(B, S, 1)
    return pl.pallas_call(
        _body,
        grid=(n_s, B),
        in_specs=[
            pl.BlockSpec((1, BS, 1), lambda s, b: (b, s, 0)),
            pl.BlockSpec((1, BS, D), lambda s, b: (b, s, 0)),
            pl.BlockSpec((BS, D), lambda s, b: (s, 0)),
        ],
        out_specs=pl.BlockSpec((1, BS, D), lambda s, b: (b, s, 0)),
        out_shape=jax.ShapeDtypeStruct((B, S, D), jnp.float32),
    )(sym3, embedded, pe)
